# baseline (speedup 1.0000x reference)
import functools

import jax
import jax.numpy as jnp
from jax import lax
from jax.experimental import pallas as pl
from jax.experimental.pallas import tpu as pltpu
from jax.experimental.pallas import tpu_sc as plsc

N_NODES = 10000
N_EDGES = 320000
IN_FEAT = 128
OUT_FEAT = 128
NUM_BASES = 4
NUM_RELS = 8

NC = 2
NS = 16
NW = NC * NS
B = 128
NBLK = 80
EPW = NBLK * B
E_PAD = NW * EPW
ROWS_PER_TILE = 632
N_PAD = ROWS_PER_TILE * NS
L = 16


def _xw_body(x_ref, weff_ref, out_ref):
    out_ref[...] = jnp.dot(x_ref[...], weff_ref[0],
                           preferred_element_type=jnp.float32)


def _tc_transform(x, weff):
    BN = 1000
    grid = (NUM_RELS, N_NODES // BN)
    return pl.pallas_call(
        _xw_body,
        grid=grid,
        in_specs=[
            pl.BlockSpec((BN, IN_FEAT), lambda r, nb: (nb, 0)),
            pl.BlockSpec((1, IN_FEAT, OUT_FEAT), lambda r, nb: (r, 0, 0)),
        ],
        out_specs=pl.BlockSpec(
            (BN, OUT_FEAT),
            lambda r, nb: (r * (N_NODES // BN) + nb, 0)),
        out_shape=jax.ShapeDtypeStruct((NUM_RELS * N_NODES, OUT_FEAT),
                                       jnp.float32),
    )(x, weff)


def _gidx_body(src_ref, typ_ref, out_ref):
    out_ref[...] = typ_ref[...] * N_NODES + src_ref[...]


def _tc_gidx(src, typ):
    return pl.pallas_call(
        _gidx_body,
        out_shape=jax.ShapeDtypeStruct((E_PAD // 128, 128), jnp.int32),
    )(src.reshape(E_PAD // 128, 128), typ.reshape(E_PAD // 128, 128))


def _scale_rows(rows_ref, de_ref):
    for g in range(B // L):
        ewv = lax.bitcast_convert_type(de_ref[1, pl.ds(g * L, L)],
                                       jnp.float32)
        for i16 in range(L):
            ew_b = lax.gather(
                ewv, jnp.full((L, 1), i16, jnp.int32),
                lax.GatherDimensionNumbers(
                    offset_dims=(), collapsed_slice_dims=(0,),
                    start_index_map=(0,)),
                slice_sizes=(1,),
                mode=lax.GatherScatterMode.PROMISE_IN_BOUNDS)
            i = g * L + i16
            for j in range(IN_FEAT // L):
                sl = pl.ds(j * L, L)
                rows_ref[i, sl] = rows_ref[i, sl] * ew_b


def _sc_body(xw_hbm, gidx_hbm, de_hbm, zeros_hbm, out_hbm,
             gidx_v, rows0, rows1, de0, de1, accum,
             g_sem0, g_sem1, s_sem0, s_sem1, d_sem0, d_sem1):
    c = lax.axis_index("c")
    s = lax.axis_index("s")
    wid = s * NC + c
    rows = (rows0, rows1)
    de = (de0, de1)
    g_sem = (g_sem0, g_sem1)
    s_sem = (s_sem0, s_sem1)
    d_sem = (d_sem0, d_sem1)

    row0 = s * ROWS_PER_TILE
    pltpu.sync_copy(zeros_hbm, accum.at[pl.ds(row0, ROWS_PER_TILE)])

    pltpu.sync_copy(gidx_hbm.at[wid], gidx_v)

    plsc.subcore_barrier()

    def gather_start(k, p):
        pltpu.async_copy(xw_hbm.at[gidx_v.at[k]], rows[p], g_sem[p])
        pltpu.async_copy(de_hbm.at[wid, k], de[p], d_sem[p])

    def gather_wait(k, p):
        pltpu.make_async_copy(xw_hbm.at[gidx_v.at[k]], rows[p],
                              g_sem[p]).wait()
        pltpu.make_async_copy(de_hbm.at[wid, k], de[p], d_sem[p]).wait()

    def scatter_start(k, p):
        pltpu.async_copy(rows[p], accum.at[de[p].at[0]], s_sem[p], add=True)

    def scatter_wait(k, p):
        pltpu.make_async_copy(rows[p], accum.at[de[p].at[0]],
                              s_sem[p]).wait()

    gather_start(0, 0)

    def super_iter(si, carry):
        for b in range(2):
            k = 2 * si + b
            if b == 0:
                @pl.when(si > 0)
                def _():
                    scatter_wait(k - 1, 1)
            else:
                scatter_wait(k - 1, 0)
            if b == 0:
                gather_start(k + 1, 1)
            else:
                @pl.when(si < NBLK // 2 - 1)
                def _():
                    gather_start(k + 1, 0)
            gather_wait(k, b)
            _scale_rows(rows[b], de[b])
            scatter_start(k, b)
        return carry

    lax.fori_loop(0, NBLK // 2, super_iter, 0)
    scatter_wait(NBLK - 1, 1)

    plsc.subcore_barrier()

    pltpu.sync_copy(accum.at[pl.ds(row0, ROWS_PER_TILE)],
                    out_hbm.at[c, pl.ds(row0, ROWS_PER_TILE)])


def _sc_scatter(xw, gidx, de, zeros):
    mesh = plsc.VectorSubcoreMesh(core_axis_name="c", subcore_axis_name="s",
                                  num_cores=NC, num_subcores=NS)
    f = pl.kernel(
        _sc_body,
        out_type=jax.ShapeDtypeStruct((NC, N_PAD, OUT_FEAT), jnp.float32),
        mesh=mesh,
        scratch_types=[
            pltpu.VMEM((NBLK, B), jnp.int32),
            pltpu.VMEM((B, IN_FEAT), jnp.float32),
            pltpu.VMEM((B, IN_FEAT), jnp.float32),
            pltpu.VMEM((2, B), jnp.int32),
            pltpu.VMEM((2, B), jnp.int32),
            pltpu.VMEM_SHARED((N_PAD, OUT_FEAT), jnp.float32),
            pltpu.SemaphoreType.DMA,
            pltpu.SemaphoreType.DMA,
            pltpu.SemaphoreType.DMA,
            pltpu.SemaphoreType.DMA,
            pltpu.SemaphoreType.DMA,
            pltpu.SemaphoreType.DMA,
        ],
    )
    return f(xw, gidx, de, zeros)


def _add_body(p_ref, out_ref):
    out_ref[...] = p_ref[0] + p_ref[1]


def _tc_add(partial):
    BN = 1000
    return pl.pallas_call(
        _add_body,
        grid=(N_NODES // BN,),
        in_specs=[pl.BlockSpec((NC, BN, OUT_FEAT), lambda nb: (0, nb, 0))],
        out_shape=jax.ShapeDtypeStruct((N_NODES, OUT_FEAT), jnp.float32),
        out_specs=pl.BlockSpec((BN, OUT_FEAT), lambda nb: (nb, 0)),
    )(partial)


def kernel(x, edge_index, edge_type, edge_weight, W_bases, w_comp):
    npad = E_PAD - N_EDGES
    src = jnp.pad(edge_index[0].astype(jnp.int32), (0, npad))
    typ = jnp.pad(edge_type.astype(jnp.int32), (0, npad))
    dst = jnp.pad(edge_index[1].astype(jnp.int32), (0, npad),
                  constant_values=N_NODES).reshape(NW, NBLK, B)
    ew_bits = lax.bitcast_convert_type(
        jnp.pad(edge_weight.astype(jnp.float32), (0, npad)),
        jnp.int32).reshape(NW, NBLK, B)
    de = jnp.stack([dst, ew_bits], axis=2)
    zeros = jnp.zeros((ROWS_PER_TILE, OUT_FEAT), jnp.float32)

    composed = jnp.matmul(w_comp, jnp.transpose(W_bases, (1, 0, 2)))
    weff = composed.reshape(NUM_RELS, IN_FEAT, OUT_FEAT)

    gidx = _tc_gidx(src, typ).reshape(NW, NBLK, B)
    xw = _tc_transform(x, weff)
    partial = _sc_scatter(xw, gidx, de, zeros)
    return _tc_add(partial)


# speedup vs baseline: 1.7544x; 1.7544x over previous
import jax
import jax.numpy as jnp
import numpy as np
from jax import lax
from jax.experimental import pallas as pl
from jax.experimental.pallas import tpu as pltpu
from jax.experimental.pallas import tpu_sc as plsc

N_NODES = 10000
N_EDGES = 320000
IN_FEAT = 128
OUT_FEAT = 128
NUM_BASES = 4
NUM_RELS = 8

NC = 2
NS = 16
NW = NC * NS
B = 128
NB0 = 100
NB1 = 57
E_PAD = NS * (NB0 + NB1) * B
ROWS_PER_TILE = 632
N_PAD = ROWS_PER_TILE * NS
L = 16


def _xw_body(x_ref, weff_ref, out_ref):
    out_ref[...] = jnp.dot(x_ref[...], weff_ref[0],
                           preferred_element_type=jnp.float32)


def _tc_transform(x, weff):
    BN = 2000
    grid = (NUM_RELS, N_NODES // BN)
    return pl.pallas_call(
        _xw_body,
        grid=grid,
        in_specs=[
            pl.BlockSpec((BN, IN_FEAT), lambda r, nb: (nb, 0)),
            pl.BlockSpec((1, IN_FEAT, OUT_FEAT), lambda r, nb: (r, 0, 0)),
        ],
        out_specs=pl.BlockSpec(
            (BN, OUT_FEAT),
            lambda r, nb: (r * (N_NODES // BN) + nb, 0)),
        out_shape=jax.ShapeDtypeStruct((NUM_RELS * N_NODES, OUT_FEAT),
                                       jnp.float32),
    )(x, weff)


def _gidx_body(src_ref, typ_ref, out_ref):
    out_ref[...] = typ_ref[...] * N_NODES + src_ref[...]


def _tc_gidx(src, typ):
    return pl.pallas_call(
        _gidx_body,
        out_shape=jax.ShapeDtypeStruct((E_PAD // 128, 128), jnp.int32),
    )(src.reshape(E_PAD // 128, 128), typ.reshape(E_PAD // 128, 128))


def _broadcast_lane(vec, i16):
    return lax.gather(
        vec, jnp.full((L, 1), i16, jnp.int32),
        lax.GatherDimensionNumbers(
            offset_dims=(), collapsed_slice_dims=(0,),
            start_index_map=(0,)),
        slice_sizes=(1,),
        mode=lax.GatherScatterMode.PROMISE_IN_BOUNDS)


def _sc_body(xw_hbm, gidx_hbm, dstew_hbm, zeros_hbm, out_hbm,
             gidx_v, dstew_v, dstbuf, rows_v, accum):
    c = lax.axis_index("c")
    s = lax.axis_index("s")
    wid = s * NC + c

    row0 = s * ROWS_PER_TILE
    pltpu.sync_copy(zeros_hbm, accum.at[pl.ds(row0, ROWS_PER_TILE)])

    pltpu.sync_copy(gidx_hbm.at[wid], gidx_v)
    pltpu.sync_copy(dstew_hbm.at[wid], dstew_v)

    plsc.subcore_barrier()

    def block(k, carry):
        pltpu.sync_copy(xw_hbm.at[gidx_v.at[k]], rows_v)

        for g in range(B // L):
            w = dstew_v[k, pl.ds(g * L, L)]
            dstbuf[pl.ds(g * L, L)] = w & 16383
            ewv = lax.bitcast_convert_type(w & jnp.int32(-16384),
                                           jnp.float32)
            for i16 in range(L):
                ew_b = _broadcast_lane(ewv, i16)
                i = g * L + i16
                for j in range(IN_FEAT // L):
                    sl = pl.ds(j * L, L)
                    rows_v[i, sl] = rows_v[i, sl] * ew_b

        pltpu.sync_copy(rows_v, accum.at[dstbuf], add=True)
        return carry

    nblk = jnp.where(c == 0, NB0, NB1)
    lax.fori_loop(0, nblk, block, 0)

    plsc.subcore_barrier()

    pltpu.sync_copy(accum.at[pl.ds(row0, ROWS_PER_TILE)],
                    out_hbm.at[c, pl.ds(row0, ROWS_PER_TILE)])


def _sc_scatter(xw, gidx, dstew, zeros):
    mesh = plsc.VectorSubcoreMesh(core_axis_name="c", subcore_axis_name="s",
                                  num_cores=NC, num_subcores=NS)
    f = pl.kernel(
        _sc_body,
        out_type=jax.ShapeDtypeStruct((NC, N_PAD, OUT_FEAT), jnp.float32),
        mesh=mesh,
        scratch_types=[
            pltpu.VMEM((NB0, B), jnp.int32),
            pltpu.VMEM((NB0, B), jnp.int32),
            pltpu.VMEM((B,), jnp.int32),
            pltpu.VMEM((B, IN_FEAT), jnp.float32),
            pltpu.VMEM_SHARED((N_PAD, OUT_FEAT), jnp.float32),
        ],
    )
    return f(xw, gidx, dstew, zeros)


def _add_body(p_ref, out_ref):
    out_ref[...] = p_ref[0] + p_ref[1]


def _tc_add(partial):
    BN = 1000
    return pl.pallas_call(
        _add_body,
        grid=(N_NODES // BN,),
        in_specs=[pl.BlockSpec((NC, BN, OUT_FEAT), lambda nb: (0, nb, 0))],
        out_shape=jax.ShapeDtypeStruct((N_NODES, OUT_FEAT), jnp.float32),
        out_specs=pl.BlockSpec((BN, OUT_FEAT), lambda nb: (nb, 0)),
    )(partial)


def _to_worker_slabs(flat):
    n0 = NS * NB0 * B
    part0 = flat[:n0].reshape(NS, NB0 * B)
    part1 = flat[n0:].reshape(NS, NB1 * B)
    part1 = jnp.pad(part1, ((0, 0), (0, (NB0 - NB1) * B)))
    return jnp.stack([part0, part1], axis=1).reshape(NW, NB0, B)


def kernel(x, edge_index, edge_type, edge_weight, W_bases, w_comp):
    npad = E_PAD - N_EDGES
    src = jnp.pad(edge_index[0].astype(jnp.int32), (0, npad))
    typ = jnp.pad(edge_type.astype(jnp.int32), (0, npad))
    dst = jnp.pad(edge_index[1].astype(jnp.int32), (0, npad),
                  constant_values=N_NODES)
    ew_bits = lax.bitcast_convert_type(
        jnp.pad(edge_weight.astype(jnp.float32), (0, npad)), jnp.int32)
    dstew = _to_worker_slabs((ew_bits & jnp.int32(-16384)) | dst)
    zeros = jnp.zeros((ROWS_PER_TILE, OUT_FEAT), jnp.float32)

    composed = jnp.matmul(w_comp, jnp.transpose(W_bases, (1, 0, 2)))
    weff = composed.reshape(NUM_RELS, IN_FEAT, OUT_FEAT)

    gidx = _to_worker_slabs(_tc_gidx(src, typ).reshape(-1))
    xw = _tc_transform(x, weff)
    partial = _sc_scatter(xw, gidx, dstew, zeros)
    return _tc_add(partial)
